# Initial kernel scaffold; baseline (speedup 1.0000x reference)
import jax
import jax.numpy as jnp
from jax.experimental import pallas as pl


def kernel(input_ids, emb_table):
    raise NotImplementedError("write your pallas kernel here")


# speedup vs baseline: 2.4441x; 2.4441x over previous
import functools

import jax
import jax.numpy as jnp
from jax import lax
from jax.experimental import pallas as pl
from jax.experimental.pallas import tpu as pltpu
from jax.experimental.pallas import tpu_sc as plsc

B = 4096
S = 200
V = 23
H = 4

NC = 2
NS = 16
NW = NC * NS

NIDS = B * S
IDS_PER_W = NIDS // NW
PIECE = 3200
NPIECES = IDS_PER_W // PIECE
VREGS_PER_PIECE = PIECE * H // 16

TAB_PAD = 96

LOGIT_ROW = S * V
HOT = (S - 1) * V + 7


def _sc_body(ids_hbm, tab_hbm, hid_hbm, hid2_hbm,
             ids_v, hid_v, hid2_v, tab_v):
    cid = lax.axis_index("c")
    sid = lax.axis_index("s")
    wid = sid * NC + cid

    iota = lax.iota(jnp.int32, 16)
    idiv = iota >> 2
    imod = iota & 3
    quarter = jnp.float32(0.25)

    pltpu.sync_copy(tab_hbm, tab_v)

    wbase = wid * IDS_PER_W
    for p in range(NPIECES):
        pbase = wbase + p * PIECE
        pltpu.sync_copy(ids_hbm.at[pl.ds(pbase, PIECE)], ids_v)

        def gbody(j, _):
            i1 = j * 4 + idiv
            g1 = plsc.load_gather(ids_v, [i1])
            i2 = g1 * 4 + imod
            g2 = plsc.load_gather(tab_v, [i2])
            hid_v[pl.ds(j * 16, 16)] = g2
            hid2_v[pl.ds(j * 16, 16)] = g2 + quarter
            return 0

        lax.fori_loop(0, VREGS_PER_PIECE, gbody, 0, unroll=8)

        obase = pbase * H
        pltpu.sync_copy(hid_v, hid_hbm.at[pl.ds(obase, PIECE * H)])
        pltpu.sync_copy(hid2_v, hid2_hbm.at[pl.ds(obase, PIECE * H)])


@functools.lru_cache(maxsize=None)
def _make_sc_call(interpret=False):
    mesh = plsc.VectorSubcoreMesh(
        core_axis_name="c", subcore_axis_name="s",
        num_cores=NC, num_subcores=NS)
    return pl.kernel(
        _sc_body,
        out_type=[
            jax.ShapeDtypeStruct((NIDS * H,), jnp.float32),
            jax.ShapeDtypeStruct((NIDS * H,), jnp.float32),
        ],
        mesh=mesh,
        scratch_types=[
            pltpu.VMEM((PIECE,), jnp.int32),
            pltpu.VMEM((PIECE * H,), jnp.float32),
            pltpu.VMEM((PIECE * H,), jnp.float32),
            pltpu.VMEM((TAB_PAD,), jnp.float32),
        ],
        compiler_params=pltpu.CompilerParams(needs_layout_passes=False),
        interpret=interpret,
    )


def _logits_body(out_ref):
    r = out_ref.shape[0]
    lane = lax.broadcasted_iota(jnp.int32, (r, LOGIT_ROW), 1)
    out_ref[...] = jnp.where(lane == HOT, jnp.float32(10.0), jnp.float32(0.0))


_LROWS = 128


def _logits_call():
    return pl.pallas_call(
        _logits_body,
        grid=(B // _LROWS,),
        out_specs=pl.BlockSpec((_LROWS, LOGIT_ROW), lambda i: (i, 0)),
        out_shape=jax.ShapeDtypeStruct((B, LOGIT_ROW), jnp.float32),
    )


@jax.jit
def kernel(input_ids, emb_table):
    ids_flat = input_ids.reshape(-1)
    tab_flat = jnp.zeros((TAB_PAD,), jnp.float32).at[: V * H].set(
        emb_table.reshape(-1))
    hid_flat, hid2_flat = _make_sc_call()(ids_flat, tab_flat)
    logits2d = _logits_call()()
    hidden = hid_flat.reshape(B, S, H)
    hidden2 = hid2_flat.reshape(B, S, H)
    logits = logits2d.reshape(B, S, V)
    return (hidden, hidden2, logits)


# speedup vs baseline: 26.8586x; 10.9892x over previous
import functools

import jax
import jax.numpy as jnp
from jax import lax
from jax.experimental import pallas as pl
from jax.experimental.pallas import tpu as pltpu
from jax.experimental.pallas import tpu_sc as plsc

B = 4096
S = 200
V = 23
H = 4

NC = 2
NS = 16
NW = NC * NS

TAB_PAD = 96

ST = S // 8
BT = B // 128
BTG = 8
NSUB = ST * (BT // BTG)
SUB_PER_W = -(-NSUB // NW)
IDS_PER_SUB = BTG * 8 * 128

LOGIT_ROW = S * V


def _sc_body(ids_hbm, tab_hbm, hid_hbm, hid2_hbm,
             ids_v, hid_v, hid2_v, tab_v):
    cid = lax.axis_index("c")
    sid = lax.axis_index("s")
    wid = sid * NC + cid

    quarter = jnp.float32(0.25)
    pltpu.sync_copy(tab_hbm, tab_v)

    for k in range(SUB_PER_W):
        sub = k * NW + wid

        @pl.when(sub < NSUB)
        def _():
            st = sub // (BT // BTG)
            btg = sub % (BT // BTG)
            in_off = st * (BT * 1024) + btg * (BTG * 1024)
            pltpu.sync_copy(ids_hbm.at[pl.ds(in_off, IDS_PER_SUB)], ids_v)

            def gbody(m, _):
                btr = m >> 6
                sl = (m >> 3) & 7
                bg = m & 7
                idv = ids_v[pl.ds(btr * 1024 + sl * 128 + bg * 16, 16)]
                i0 = idv * 4
                r0 = btr * 4
                c0 = bg * 16
                for h in range(H):
                    g = plsc.load_gather(tab_v, [i0 + h])
                    hid_v[sl, r0 + h, pl.ds(c0, 16)] = g
                    hid2_v[sl, r0 + h, pl.ds(c0, 16)] = g + quarter
                return 0

            lax.fori_loop(0, 8 * 8 * 8, gbody, 0, unroll=4)

            for sl in range(8):
                s = st * 8 + sl
                dst = pl.ds(btg * (BTG * 4), BTG * 4)
                pltpu.sync_copy(hid_v.at[sl], hid_hbm.at[s, dst])
                pltpu.sync_copy(hid2_v.at[sl], hid2_hbm.at[s, dst])


@functools.lru_cache(maxsize=None)
def _make_sc_call():
    mesh = plsc.VectorSubcoreMesh(
        core_axis_name="c", subcore_axis_name="s",
        num_cores=NC, num_subcores=NS)
    return pl.kernel(
        _sc_body,
        out_type=[
            jax.ShapeDtypeStruct((S, 128, 128), jnp.float32),
            jax.ShapeDtypeStruct((S, 128, 128), jnp.float32),
        ],
        mesh=mesh,
        scratch_types=[
            pltpu.VMEM((IDS_PER_SUB,), jnp.int32),
            pltpu.VMEM((8, BTG * 4, 128), jnp.float32),
            pltpu.VMEM((8, BTG * 4, 128), jnp.float32),
            pltpu.VMEM((TAB_PAD,), jnp.float32),
        ],
        compiler_params=pltpu.CompilerParams(needs_layout_passes=False),
    )


def _logits_body(out_ref):
    vblk = pl.program_id(0)
    s_iota = lax.broadcasted_iota(jnp.int32, out_ref.shape, 1)
    hot = jnp.logical_and(vblk == 7, s_iota == S - 1)
    out_ref[...] = jnp.where(hot, jnp.float32(10.0), jnp.float32(0.0))


def _logits_call():
    return pl.pallas_call(
        _logits_body,
        grid=(V, 8),
        out_specs=pl.BlockSpec((1, S, 512), lambda v, b: (v, 0, b)),
        out_shape=jax.ShapeDtypeStruct((V, S, B), jnp.float32),
    )


@jax.jit
def kernel(input_ids, emb_table):
    ids_lin = (input_ids.transpose(1, 0)
               .reshape(ST, 8, BT, 128)
               .transpose(0, 2, 1, 3)
               .reshape(B * S))
    tab_flat = jnp.zeros((TAB_PAD,), jnp.float32).at[: V * H].set(
        emb_table.reshape(-1))
    hid_lin, hid2_lin = _make_sc_call()(ids_lin, tab_flat)
    logits_t = _logits_call()()

    def unbitcast(y):
        return (y.reshape(S, BT, H, 128).transpose(1, 3, 0, 2)
                .reshape(B, S, H))

    return (unbitcast(hid_lin), unbitcast(hid2_lin),
            logits_t.transpose(2, 1, 0))


# speedup vs baseline: 30.1950x; 1.1242x over previous
import functools

import jax
import jax.numpy as jnp
from jax import lax
from jax.experimental import pallas as pl
from jax.experimental.pallas import tpu as pltpu
from jax.experimental.pallas import tpu_sc as plsc

B = 4096
S = 200
V = 23
H = 4

NC = 2
NS = 16
NW = NC * NS

TAB_PAD = 96

ST = S // 8
BT = B // 128
BTG = 4
NSUB = ST * (BT // BTG)
SUB_PER_W = -(-NSUB // NW)
IDS_PER_SUB = BTG * 8 * 128

LOGIT_ROW = S * V


def _sc_body(ids_hbm, tab_hbm, hid_hbm, hid2_hbm,
             ids_v, hid_v, hid2_v, tab_v, in_sem, out_sem):
    cid = lax.axis_index("c")
    sid = lax.axis_index("s")
    wid = sid * NC + cid

    quarter = jnp.float32(0.25)
    pltpu.sync_copy(tab_hbm, tab_v)

    def in_off(sub):
        st = sub // (BT // BTG)
        btg = sub % (BT // BTG)
        return st * (BT * 1024) + btg * (IDS_PER_SUB)

    @pl.when(wid < NSUB)
    def _():
        pltpu.async_copy(
            ids_hbm.at[pl.ds(in_off(wid), IDS_PER_SUB)],
            ids_v.at[0], in_sem)

    for k in range(SUB_PER_W):
        sub = k * NW + wid
        buf = k % 2

        @pl.when(sub < NSUB)
        def _():
            st = sub // (BT // BTG)
            btg = sub % (BT // BTG)
            pltpu.make_async_copy(
                ids_hbm.at[pl.ds(in_off(sub), IDS_PER_SUB)],
                ids_v.at[buf], in_sem).wait()
            if k + 1 < SUB_PER_W:
                nxt = sub + NW

                @pl.when(nxt < NSUB)
                def _():
                    pltpu.async_copy(
                        ids_hbm.at[pl.ds(in_off(nxt), IDS_PER_SUB)],
                        ids_v.at[1 - buf], in_sem)

            def gbody(m, _):
                btr = m >> 6
                sl = (m >> 3) & 7
                bg = m & 7
                idv = ids_v[buf, pl.ds(m * 16, 16)]
                i0 = idv * 4
                r0 = btr * 4
                c0 = bg * 16
                for h in range(H):
                    g = plsc.load_gather(tab_v, [i0 + h])
                    hid_v[sl, r0 + h, pl.ds(c0, 16)] = g
                    hid2_v[sl, r0 + h, pl.ds(c0, 16)] = g + quarter
                return 0

            lax.fori_loop(0, BTG * 8 * 8, gbody, 0, unroll=8)

            copies = []
            for sl in range(8):
                s = st * 8 + sl
                dst = pl.ds(btg * (BTG * 4), BTG * 4)
                c1 = pltpu.async_copy(hid_v.at[sl], hid_hbm.at[s, dst],
                                      out_sem)
                c2 = pltpu.async_copy(hid2_v.at[sl], hid2_hbm.at[s, dst],
                                      out_sem)
                copies += [c1, c2]
            for c in copies:
                c.wait()


@functools.lru_cache(maxsize=None)
def _make_sc_call():
    mesh = plsc.VectorSubcoreMesh(
        core_axis_name="c", subcore_axis_name="s",
        num_cores=NC, num_subcores=NS)
    return pl.kernel(
        _sc_body,
        out_type=[
            jax.ShapeDtypeStruct((S, 128, 128), jnp.float32),
            jax.ShapeDtypeStruct((S, 128, 128), jnp.float32),
        ],
        mesh=mesh,
        scratch_types=[
            pltpu.VMEM((2, IDS_PER_SUB), jnp.int32),
            pltpu.VMEM((8, BTG * 4, 128), jnp.float32),
            pltpu.VMEM((8, BTG * 4, 128), jnp.float32),
            pltpu.VMEM((TAB_PAD,), jnp.float32),
            pltpu.SemaphoreType.DMA,
            pltpu.SemaphoreType.DMA,
        ],
        compiler_params=pltpu.CompilerParams(needs_layout_passes=False),
    )


def _logits_body(out_ref):
    vblk = pl.program_id(0)
    s_iota = lax.broadcasted_iota(jnp.int32, out_ref.shape, 1)
    hot = jnp.logical_and(vblk == 7, s_iota == S - 1)
    out_ref[...] = jnp.where(hot, jnp.float32(10.0), jnp.float32(0.0))


def _logits_call():
    return pl.pallas_call(
        _logits_body,
        grid=(V, 8),
        out_specs=pl.BlockSpec((1, S, 512), lambda v, b: (v, 0, b)),
        out_shape=jax.ShapeDtypeStruct((V, S, B), jnp.float32),
    )


@jax.jit
def kernel(input_ids, emb_table):
    ids_lin = (input_ids.transpose(1, 0)
               .reshape(ST, 8, BT, 128)
               .transpose(0, 2, 1, 3)
               .reshape(B * S))
    tab_flat = jnp.zeros((TAB_PAD,), jnp.float32).at[: V * H].set(
        emb_table.reshape(-1))
    hid_lin, hid2_lin = _make_sc_call()(ids_lin, tab_flat)
    logits_t = _logits_call()()

    def unbitcast(y):
        return (y.reshape(S, BT, H, 128).transpose(1, 3, 0, 2)
                .reshape(B, S, H))

    return (unbitcast(hid_lin), unbitcast(hid2_lin),
            logits_t.transpose(2, 1, 0))
